# speedup vs baseline: 9.2673x; 9.2673x over previous
import jax
import jax.numpy as jnp
from jax import lax
from jax.experimental import pallas as pl
from jax.experimental.pallas import tpu as pltpu
from jax.experimental.pallas import tpu_sc as plsc

NUM_VOCAB = 100000
EMBED_DIM = 128
BATCH = 4096
SEQ_LEN = 200

NC = 2
NS = 16
NW = NC * NS

TOTAL = BATCH * SEQ_LEN
PER_W = TOTAL // NW
CHUNK = 128
NCHUNK = PER_W // CHUNK
NBUF = 4


def _body(table_hbm, seqs_hbm, out_hbm, idx_v, rows, gsems, osems):
    wid = lax.axis_index("s") * NC + lax.axis_index("c")
    base = wid * PER_W

    pltpu.sync_copy(seqs_hbm.at[wid], idx_v)

    def gather(b, chunk):
        pltpu.async_copy(table_hbm.at[idx_v.at[chunk]], rows[b], gsems[b])

    for b in range(NBUF):
        gather(b, b)

    @pl.loop(0, NCHUNK, step=NBUF)
    def _(j):
        for b in range(NBUF):
            pltpu.make_async_copy(table_hbm.at[idx_v.at[j + b]], rows[b],
                                  gsems[b]).wait()
            pltpu.async_copy(
                rows[b], out_hbm.at[pl.ds(base + (j + b) * CHUNK, CHUNK)],
                osems[b])
        for b in range(NBUF):
            nxt = j + b + NBUF

            @pl.when(nxt < NCHUNK)
            def _():
                pltpu.make_async_copy(
                    rows[b], out_hbm.at[pl.ds(base + (j + b) * CHUNK, CHUNK)],
                    osems[b]).wait()
                gather(b, nxt)

    for b in range(NBUF):
        j0 = NCHUNK - NBUF + b
        pltpu.make_async_copy(
            rows[b], out_hbm.at[pl.ds(base + j0 * CHUNK, CHUNK)],
            osems[b]).wait()


@jax.jit
def _lookup(table, seqs_r):
    mesh = plsc.VectorSubcoreMesh(
        core_axis_name="c", subcore_axis_name="s",
        num_cores=NC, num_subcores=NS)
    fn = pl.kernel(
        _body,
        out_type=jax.ShapeDtypeStruct((TOTAL, EMBED_DIM), jnp.float32),
        mesh=mesh,
        scratch_types=dict(
            idx_v=pltpu.VMEM((NCHUNK, CHUNK), jnp.int32),
            rows=[pltpu.VMEM((CHUNK, EMBED_DIM), jnp.float32)
                  for _ in range(NBUF)],
            gsems=[pltpu.SemaphoreType.DMA for _ in range(NBUF)],
            osems=[pltpu.SemaphoreType.DMA for _ in range(NBUF)],
        ),
    )
    return fn(table, seqs_r)


def kernel(seqs, table):
    seqs_r = seqs.astype(jnp.int32).reshape(NW, NCHUNK, CHUNK)
    out = _lookup(table, seqs_r)
    return out.reshape(BATCH, SEQ_LEN, EMBED_DIM)


# speedup vs baseline: 9.2960x; 1.0031x over previous
import jax
import jax.numpy as jnp
from jax import lax
from jax.experimental import pallas as pl
from jax.experimental.pallas import tpu as pltpu
from jax.experimental.pallas import tpu_sc as plsc

NUM_VOCAB = 100000
EMBED_DIM = 128
BATCH = 4096
SEQ_LEN = 200

NC = 2
NS = 16
NW = NC * NS

TOTAL = BATCH * SEQ_LEN
PER_W = TOTAL // NW
CHUNK = 128
NCHUNK = PER_W // CHUNK
NBUF = 5


def _body(table_hbm, seqs_hbm, out_hbm, idx_v, rows, gsems, osems):
    wid = lax.axis_index("s") * NC + lax.axis_index("c")
    base = wid * PER_W

    pltpu.sync_copy(seqs_hbm.at[wid], idx_v)

    def gather(b, chunk):
        pltpu.async_copy(table_hbm.at[idx_v.at[chunk]], rows[b], gsems[b])

    for b in range(NBUF):
        gather(b, b)

    @pl.loop(0, NCHUNK, step=NBUF)
    def _(j):
        for b in range(NBUF):
            pltpu.make_async_copy(table_hbm.at[idx_v.at[j + b]], rows[b],
                                  gsems[b]).wait()
            pltpu.async_copy(
                rows[b], out_hbm.at[pl.ds(base + (j + b) * CHUNK, CHUNK)],
                osems[b])
        for b in range(NBUF):
            nxt = j + b + NBUF

            @pl.when(nxt < NCHUNK)
            def _():
                pltpu.make_async_copy(
                    rows[b], out_hbm.at[pl.ds(base + (j + b) * CHUNK, CHUNK)],
                    osems[b]).wait()
                gather(b, nxt)

    for b in range(NBUF):
        j0 = NCHUNK - NBUF + b
        pltpu.make_async_copy(
            rows[b], out_hbm.at[pl.ds(base + j0 * CHUNK, CHUNK)],
            osems[b]).wait()


@jax.jit
def _lookup(table, seqs_r):
    mesh = plsc.VectorSubcoreMesh(
        core_axis_name="c", subcore_axis_name="s",
        num_cores=NC, num_subcores=NS)
    fn = pl.kernel(
        _body,
        out_type=jax.ShapeDtypeStruct((TOTAL, EMBED_DIM), jnp.float32),
        mesh=mesh,
        scratch_types=dict(
            idx_v=pltpu.VMEM((NCHUNK, CHUNK), jnp.int32),
            rows=[pltpu.VMEM((CHUNK, EMBED_DIM), jnp.float32)
                  for _ in range(NBUF)],
            gsems=[pltpu.SemaphoreType.DMA for _ in range(NBUF)],
            osems=[pltpu.SemaphoreType.DMA for _ in range(NBUF)],
        ),
    )
    return fn(table, seqs_r)


def kernel(seqs, table):
    seqs_r = seqs.astype(jnp.int32).reshape(NW, NCHUNK, CHUNK)
    out = _lookup(table, seqs_r)
    return out.reshape(BATCH, SEQ_LEN, EMBED_DIM)
